# baseline (speedup 1.0000x reference)
import jax
import jax.numpy as jnp
from jax.experimental import pallas as pl
from jax.experimental.pallas import tpu as pltpu

_NL = 127.0
_EPS = 1e-8


def _absmax_kernel(x_ref, o_ref):
    o_ref[0, 0, :] = jnp.broadcast_to(jnp.max(jnp.abs(x_ref[...])), (128,))


def _collapse_kernel(part_ref, w0_ref, w0t_ref, b0_ref, w1_ref, w1t_ref,
                     b1_ref, w2_ref, w2t_ref, b2_ref, w3_ref, w3t_ref,
                     b3_ref, a_ref, d_ref, rs_ref):
    s = jnp.maximum(jnp.max(part_ref[...]) * (1.0 / _NL), _EPS)

    def qw(w_ref):
        w = w_ref[...]
        sf = jnp.maximum(
            jnp.max(jnp.abs(w), axis=1, keepdims=True) * (1.0 / _NL), _EPS)
        return jnp.clip(jnp.round(w / sf), -_NL, _NL), sf

    def sf_cols(wt_ref):
        wt = wt_ref[...]
        return jnp.maximum(
            jnp.max(jnp.abs(wt), axis=0, keepdims=True) * (1.0 / _NL), _EPS)

    def qb(b_ref, sf):
        return jnp.clip(jnp.round(b_ref[...] / (sf * s)), -_NL, _NL)

    w0q, sf0 = qw(w0_ref)
    acc_m, acc_d = w0q, qb(b0_ref, sf0)
    prevt = (w0t_ref, w1t_ref, w2t_ref)
    for li, (w_ref, b_ref) in enumerate(
            ((w1_ref, b1_ref), (w2_ref, b2_ref), (w3_ref, b3_ref))):
        wq, sfl = qw(w_ref)
        we = wq * sf_cols(prevt[li])
        acc_m = jnp.dot(we, acc_m, preferred_element_type=jnp.float32)
        acc_d = (jnp.dot(we, acc_d, preferred_element_type=jnp.float32)
                 + qb(b_ref, sfl))
        sf_last = sfl
    bsf = sf_last * s
    a_ref[...] = acc_m * bsf
    d_ref[...] = acc_d * bsf
    rs_ref[...] = jnp.reshape(1.0 / s, (1, 1))


def _apply_kernel(a_ref, d_ref, rs_ref, x_ref, o_ref):
    z = jnp.clip(jnp.round(x_ref[...] * rs_ref[0, 0]), -_NL, _NL)
    o_ref[...] = (jnp.dot(a_ref[...], z, preferred_element_type=jnp.float32)
                  + d_ref[...])


def kernel(x, w0, b0, w2, b2, w4, b4, w6, b6):
    B = x.shape[0]
    xt = x.T
    n_out = w6.shape[0]

    G = 4 if B % 4 == 0 else 1
    part = pl.pallas_call(
        _absmax_kernel,
        grid=(G,),
        in_specs=[pl.BlockSpec((16, B // G), lambda i: (0, i))],
        out_specs=pl.BlockSpec((1, 1, 128), lambda i: (i, 0, 0)),
        out_shape=jax.ShapeDtypeStruct((G, 1, 128), jnp.float32),
        compiler_params=pltpu.CompilerParams(
            dimension_semantics=("parallel",)),
    )(xt)

    full = lambda shape: pl.BlockSpec(shape, lambda *_: (0,) * len(shape))
    c_specs = [full((G, 1, 128))]
    c_ops = [part]
    for w, b in ((w0, b0), (w2, b2), (w4, b4), (w6, b6)):
        c_specs += [full(w.shape), full(w.T.shape), full((w.shape[0], 1))]
        c_ops += [w, w.T, b[:, None]]
    a_mat, d_vec, rs = pl.pallas_call(
        _collapse_kernel,
        in_specs=c_specs,
        out_specs=(full((n_out, 16)), full((n_out, 1)), full((1, 1))),
        out_shape=(jax.ShapeDtypeStruct((n_out, 16), jnp.float32),
                   jax.ShapeDtypeStruct((n_out, 1), jnp.float32),
                   jax.ShapeDtypeStruct((1, 1), jnp.float32)),
    )(*c_ops)

    cblk = 262144 if B % 262144 == 0 else B
    outt = pl.pallas_call(
        _apply_kernel,
        grid=(B // cblk,),
        in_specs=[full((n_out, 16)), full((n_out, 1)), full((1, 1)),
                  pl.BlockSpec((16, cblk), lambda i: (0, i))],
        out_specs=pl.BlockSpec((n_out, cblk), lambda i: (0, i)),
        out_shape=jax.ShapeDtypeStruct((n_out, B), jnp.float32),
        compiler_params=pltpu.CompilerParams(
            dimension_semantics=("parallel",)),
    )(a_mat, d_vec, rs, xt)
    return outt.T


# speedup vs baseline: 1.0569x; 1.0569x over previous
import jax
import jax.numpy as jnp
from jax.experimental import pallas as pl
from jax.experimental.pallas import tpu as pltpu

_NL = 127.0
_EPS = 1e-8


def _absmax_kernel(x_ref, o_ref):
    o_ref[0, 0, :] = jnp.broadcast_to(jnp.max(jnp.abs(x_ref[...])), (128,))


def _collapse_kernel(part_ref, w0_ref, w0t_ref, b0_ref, w1_ref, w1t_ref,
                     b1_ref, w2_ref, w2t_ref, b2_ref, w3_ref, w3t_ref,
                     b3_ref, a_ref, d_ref, rs_ref):
    s = jnp.maximum(jnp.max(part_ref[...]) * (1.0 / _NL), _EPS)

    def qw(w_ref):
        w = w_ref[...]
        sf = jnp.maximum(
            jnp.max(jnp.abs(w), axis=1, keepdims=True) * (1.0 / _NL), _EPS)
        return jnp.clip(jnp.round(w / sf), -_NL, _NL), sf

    def sf_cols(wt_ref):
        wt = wt_ref[...]
        return jnp.maximum(
            jnp.max(jnp.abs(wt), axis=0, keepdims=True) * (1.0 / _NL), _EPS)

    def qb(b_ref, sf):
        return jnp.clip(jnp.round(b_ref[...] / (sf * s)), -_NL, _NL)

    w0q, sf0 = qw(w0_ref)
    acc_m, acc_d = w0q, qb(b0_ref, sf0)
    prevt = (w0t_ref, w1t_ref, w2t_ref)
    for li, (w_ref, b_ref) in enumerate(
            ((w1_ref, b1_ref), (w2_ref, b2_ref), (w3_ref, b3_ref))):
        wq, sfl = qw(w_ref)
        we = wq * sf_cols(prevt[li])
        acc_m = jnp.dot(we, acc_m, preferred_element_type=jnp.float32)
        acc_d = (jnp.dot(we, acc_d, preferred_element_type=jnp.float32)
                 + qb(b_ref, sfl))
        sf_last = sfl
    bsf = sf_last * s
    a_ref[...] = acc_m * bsf
    d_ref[...] = acc_d * bsf
    rs_ref[...] = jnp.reshape(1.0 / s, (1, 1))


def _apply_kernel(a_ref, d_ref, rs_ref, x_ref, o_ref):
    z = jnp.clip(jnp.round(x_ref[...] * rs_ref[0, 0]), -_NL, _NL)
    o_ref[...] = (jnp.dot(a_ref[...], z, preferred_element_type=jnp.float32)
                  + d_ref[...])


def kernel(x, w0, b0, w2, b2, w4, b4, w6, b6):
    B = x.shape[0]
    xt = x.T
    n_out = w6.shape[0]

    G = 8 if B % 8 == 0 else 1
    part = pl.pallas_call(
        _absmax_kernel,
        grid=(G,),
        in_specs=[pl.BlockSpec((16, B // G), lambda i: (0, i))],
        out_specs=pl.BlockSpec((1, 1, 128), lambda i: (i, 0, 0)),
        out_shape=jax.ShapeDtypeStruct((G, 1, 128), jnp.float32),
        compiler_params=pltpu.CompilerParams(
            dimension_semantics=("parallel",)),
    )(xt)

    full = lambda shape: pl.BlockSpec(shape, lambda *_: (0,) * len(shape))
    c_specs = [full((G, 1, 128))]
    c_ops = [part]
    for w, b in ((w0, b0), (w2, b2), (w4, b4), (w6, b6)):
        c_specs += [full(w.shape), full(w.T.shape), full((w.shape[0], 1))]
        c_ops += [w, w.T, b[:, None]]
    a_mat, d_vec, rs = pl.pallas_call(
        _collapse_kernel,
        in_specs=c_specs,
        out_specs=(full((n_out, 16)), full((n_out, 1)), full((1, 1))),
        out_shape=(jax.ShapeDtypeStruct((n_out, 16), jnp.float32),
                   jax.ShapeDtypeStruct((n_out, 1), jnp.float32),
                   jax.ShapeDtypeStruct((1, 1), jnp.float32)),
    )(*c_ops)

    cblk = 131072 if B % 131072 == 0 else B
    outt = pl.pallas_call(
        _apply_kernel,
        grid=(B // cblk,),
        in_specs=[full((n_out, 16)), full((n_out, 1)), full((1, 1)),
                  pl.BlockSpec((16, cblk), lambda i: (0, i))],
        out_specs=pl.BlockSpec((n_out, cblk), lambda i: (0, i)),
        out_shape=jax.ShapeDtypeStruct((n_out, B), jnp.float32),
        compiler_params=pltpu.CompilerParams(
            dimension_semantics=("parallel",)),
    )(a_mat, d_vec, rs, xt)
    return outt.T


# speedup vs baseline: 1.0610x; 1.0039x over previous
import jax
import jax.numpy as jnp
from jax.experimental import pallas as pl
from jax.experimental.pallas import tpu as pltpu

_NL = 127.0
_EPS = 1e-8


def _absmax_kernel(x_ref, o_ref):
    o_ref[0, 0, :] = jnp.broadcast_to(jnp.max(jnp.abs(x_ref[...])), (128,))


def _collapse_kernel(part_ref, w0_ref, w0t_ref, b0_ref, w1_ref, w1t_ref,
                     b1_ref, w2_ref, w2t_ref, b2_ref, w3_ref, w3t_ref,
                     b3_ref, a_ref, d_ref, rs_ref):
    s = jnp.maximum(jnp.max(part_ref[...]) * (1.0 / _NL), _EPS)

    def qw(w_ref):
        w = w_ref[...]
        sf = jnp.maximum(
            jnp.max(jnp.abs(w), axis=1, keepdims=True) * (1.0 / _NL), _EPS)
        return jnp.clip(jnp.round(w / sf), -_NL, _NL), sf

    def sf_cols(wt_ref):
        wt = wt_ref[...]
        return jnp.maximum(
            jnp.max(jnp.abs(wt), axis=0, keepdims=True) * (1.0 / _NL), _EPS)

    def qb(b_ref, sf):
        return jnp.clip(jnp.round(b_ref[...] / (sf * s)), -_NL, _NL)

    w0q, sf0 = qw(w0_ref)
    acc_m, acc_d = w0q, qb(b0_ref, sf0)
    prevt = (w0t_ref, w1t_ref, w2t_ref)
    for li, (w_ref, b_ref) in enumerate(
            ((w1_ref, b1_ref), (w2_ref, b2_ref), (w3_ref, b3_ref))):
        wq, sfl = qw(w_ref)
        we = wq * sf_cols(prevt[li])
        acc_m = jnp.dot(we, acc_m, preferred_element_type=jnp.float32)
        acc_d = (jnp.dot(we, acc_d, preferred_element_type=jnp.float32)
                 + qb(b_ref, sfl))
        sf_last = sfl
    bsf = sf_last * s
    a_ref[...] = acc_m * bsf
    d_ref[...] = acc_d * bsf
    rs_ref[...] = jnp.reshape(1.0 / s, (1, 1))


def _apply_kernel(a_ref, d_ref, rs_ref, x_ref, o_ref):
    z = jnp.round(x_ref[...] * rs_ref[0, 0])
    o_ref[...] = (jnp.dot(a_ref[...], z, preferred_element_type=jnp.float32)
                  + d_ref[...])


def kernel(x, w0, b0, w2, b2, w4, b4, w6, b6):
    B = x.shape[0]
    xt = x.T
    n_out = w6.shape[0]

    G = 8 if B % 8 == 0 else 1
    part = pl.pallas_call(
        _absmax_kernel,
        grid=(G,),
        in_specs=[pl.BlockSpec((16, B // G), lambda i: (0, i))],
        out_specs=pl.BlockSpec((1, 1, 128), lambda i: (i, 0, 0)),
        out_shape=jax.ShapeDtypeStruct((G, 1, 128), jnp.float32),
        compiler_params=pltpu.CompilerParams(
            dimension_semantics=("parallel",)),
    )(xt)

    full = lambda shape: pl.BlockSpec(shape, lambda *_: (0,) * len(shape))
    c_specs = [full((G, 1, 128))]
    c_ops = [part]
    for w, b in ((w0, b0), (w2, b2), (w4, b4), (w6, b6)):
        c_specs += [full(w.shape), full(w.T.shape), full((w.shape[0], 1))]
        c_ops += [w, w.T, b[:, None]]
    a_mat, d_vec, rs = pl.pallas_call(
        _collapse_kernel,
        in_specs=c_specs,
        out_specs=(full((n_out, 16)), full((n_out, 1)), full((1, 1))),
        out_shape=(jax.ShapeDtypeStruct((n_out, 16), jnp.float32),
                   jax.ShapeDtypeStruct((n_out, 1), jnp.float32),
                   jax.ShapeDtypeStruct((1, 1), jnp.float32)),
    )(*c_ops)

    cblk = 131072 if B % 131072 == 0 else B
    outt = pl.pallas_call(
        _apply_kernel,
        grid=(B // cblk,),
        in_specs=[full((n_out, 16)), full((n_out, 1)), full((1, 1)),
                  pl.BlockSpec((16, cblk), lambda i: (0, i))],
        out_specs=pl.BlockSpec((n_out, cblk), lambda i: (0, i)),
        out_shape=jax.ShapeDtypeStruct((n_out, B), jnp.float32),
        compiler_params=pltpu.CompilerParams(
            dimension_semantics=("parallel",)),
    )(a_mat, d_vec, rs, xt)
    return outt.T


# speedup vs baseline: 1.1544x; 1.0880x over previous
import jax
import jax.numpy as jnp
from jax.experimental import pallas as pl
from jax.experimental.pallas import tpu as pltpu

_NL = 127.0
_EPS = 1e-8


def _absmax_kernel(x_ref, o_ref):
    o_ref[0, 0, :] = jnp.broadcast_to(jnp.max(jnp.abs(x_ref[...])), (128,))


def _col(v):
    o = v.shape[0]
    r = jax.lax.broadcasted_iota(jnp.int32, (o, o), 0)
    c = jax.lax.broadcasted_iota(jnp.int32, (o, o), 1)
    return jnp.sum(jnp.where(r == c, v[None, :], 0.0), axis=1, keepdims=True)


def _collapse_kernel(part_ref, w0_ref, b0_ref, w1_ref, b1_ref, w2_ref,
                     b2_ref, w3_ref, b3_ref, a_ref, d_ref, rs_ref):
    s = jnp.maximum(jnp.max(part_ref[...]) * (1.0 / _NL), _EPS)

    def quant(w_ref, b_ref):
        w = w_ref[...]
        sf = jnp.maximum(
            jnp.max(jnp.abs(w), axis=1, keepdims=True) * (1.0 / _NL), _EPS)
        wq = jnp.clip(jnp.round(w / sf), -_NL, _NL)
        bq = jnp.clip(jnp.round(_col(b_ref[...]) / (sf * s)), -_NL, _NL)
        return wq, bq, sf

    def sf_row(w_ref):
        w = w_ref[...]
        return jnp.maximum(
            jnp.max(jnp.abs(w), axis=1) * (1.0 / _NL), _EPS)[None, :]

    acc_m, acc_d, sf_last = quant(w0_ref, b0_ref)
    prev = (w0_ref, w1_ref, w2_ref)
    for li, (w_ref, b_ref) in enumerate(
            ((w1_ref, b1_ref), (w2_ref, b2_ref), (w3_ref, b3_ref))):
        wq, bq, sf_last = quant(w_ref, b_ref)
        we = wq * sf_row(prev[li])
        acc_m = jnp.dot(we, acc_m, preferred_element_type=jnp.float32)
        acc_d = jnp.dot(we, acc_d, preferred_element_type=jnp.float32) + bq
    bsf = sf_last * s
    a_ref[...] = acc_m * bsf
    d_ref[...] = acc_d * bsf
    rs_ref[...] = jnp.reshape(1.0 / s, (1, 1))


def _apply_kernel(a_ref, d_ref, rs_ref, x_ref, o_ref):
    z = jnp.round(x_ref[...] * rs_ref[0, 0])
    o_ref[...] = (jnp.dot(a_ref[...], z, preferred_element_type=jnp.float32)
                  + d_ref[...])


def kernel(x, w0, b0, w2, b2, w4, b4, w6, b6):
    B = x.shape[0]
    xt = x.T
    n_out = w6.shape[0]

    G = 8 if B % 8 == 0 else 1
    part = pl.pallas_call(
        _absmax_kernel,
        grid=(G,),
        in_specs=[pl.BlockSpec((16, B // G), lambda i: (0, i))],
        out_specs=pl.BlockSpec((1, 1, 128), lambda i: (i, 0, 0)),
        out_shape=jax.ShapeDtypeStruct((G, 1, 128), jnp.float32),
        compiler_params=pltpu.CompilerParams(
            dimension_semantics=("parallel",)),
    )(xt)

    full = lambda shape: pl.BlockSpec(shape, lambda *_: (0,) * len(shape))
    c_specs = [full((G, 1, 128))]
    c_ops = [part]
    for w, b in ((w0, b0), (w2, b2), (w4, b4), (w6, b6)):
        c_specs += [full(w.shape), full(b.shape)]
        c_ops += [w, b]
    a_mat, d_vec, rs = pl.pallas_call(
        _collapse_kernel,
        in_specs=c_specs,
        out_specs=(full((n_out, 16)), full((n_out, 1)), full((1, 1))),
        out_shape=(jax.ShapeDtypeStruct((n_out, 16), jnp.float32),
                   jax.ShapeDtypeStruct((n_out, 1), jnp.float32),
                   jax.ShapeDtypeStruct((1, 1), jnp.float32)),
    )(*c_ops)

    cblk = 131072 if B % 131072 == 0 else B
    outt = pl.pallas_call(
        _apply_kernel,
        grid=(B // cblk,),
        in_specs=[full((n_out, 16)), full((n_out, 1)), full((1, 1)),
                  pl.BlockSpec((16, cblk), lambda i: (0, i))],
        out_specs=pl.BlockSpec((n_out, cblk), lambda i: (0, i)),
        out_shape=jax.ShapeDtypeStruct((n_out, B), jnp.float32),
        compiler_params=pltpu.CompilerParams(
            dimension_semantics=("parallel",)),
    )(a_mat, d_vec, rs, xt)
    return outt.T
